# Initial kernel scaffold; baseline (speedup 1.0000x reference)
import jax
import jax.numpy as jnp
from jax.experimental import pallas as pl


def kernel(name_idxs, name_len, desc_idxs, desc_len, union_idxs, union_len, price, emb_table, fc_w, fc_b):
    raise NotImplementedError("write your pallas kernel here")


# speedup vs baseline: 5.8164x; 5.8164x over previous
import functools

import jax
import jax.numpy as jnp
from jax import lax
from jax.experimental import pallas as pl
from jax.experimental.pallas import tpu as pltpu
from jax.experimental.pallas import tpu_sc as plsc

B = 4096
V = 100000
D = 64
OUT = 128
L_NAME = 20
L_DESC = 200

NC = 2
NS = 16
NW = NC * NS
BPW = B // NW
CHUNK = 112
ROWS = 2 * CHUNK
LANES = 16
NG = D // LANES


def _emb_sum_kernel(table_hbm, idx_hbm, name_out, desc_out,
                    idx_v, rows_v, nsum_v, dsum_v, sem):
    wid = lax.axis_index("s") * NC + lax.axis_index("c")
    base = wid * BPW
    pltpu.sync_copy(idx_hbm.at[pl.ds(base, BPW)], idx_v)

    def accumulate(i):
        for g in range(NG):
            sl = pl.ds(g * LANES, LANES)
            acc = rows_v[0, sl]
            for r in range(1, L_NAME):
                acc = acc + rows_v[r, sl]
            nsum_v[i, sl] = acc

        def dbody(r, accs):
            out = []
            for g in range(NG):
                a = accs[g]
                for k in range(8):
                    a = a + rows_v[L_NAME + r * 8 + k, pl.ds(g * LANES, LANES)]
                out.append(a)
            return tuple(out)

        accs = tuple(jnp.zeros((LANES,), jnp.float32) for _ in range(NG))
        accs = lax.fori_loop(0, L_DESC // 8, dbody, accs)
        for g in range(NG):
            dsum_v[i, pl.ds(g * LANES, LANES)] = accs[g]

    def body(i, _):
        c0 = pltpu.async_copy(table_hbm.at[idx_v.at[i, 0]],
                              rows_v.at[pl.ds(0, CHUNK)], sem)
        c1 = pltpu.async_copy(table_hbm.at[idx_v.at[i, 1]],
                              rows_v.at[pl.ds(CHUNK, CHUNK)], sem)
        c0.wait()
        c1.wait()
        accumulate(i)
        return 0

    lax.fori_loop(0, BPW, body, 0)

    pltpu.sync_copy(nsum_v, name_out.at[pl.ds(base, BPW)])
    pltpu.sync_copy(dsum_v, desc_out.at[pl.ds(base, BPW)])


@functools.partial(
    pl.kernel,
    out_type=(jax.ShapeDtypeStruct((B, D), jnp.float32),
              jax.ShapeDtypeStruct((B, D), jnp.float32)),
    mesh=plsc.VectorSubcoreMesh(core_axis_name="c", subcore_axis_name="s"),
    scratch_types=[
        pltpu.VMEM((BPW, 2, CHUNK), jnp.int32),
        pltpu.VMEM((ROWS, D), jnp.float32),
        pltpu.VMEM((BPW, D), jnp.float32),
        pltpu.VMEM((BPW, D), jnp.float32),
        pltpu.SemaphoreType.DMA,
    ],
    compiler_params=pltpu.CompilerParams(use_tc_tiling_on_sc=False),
)
def _emb_sums(table_hbm, idx_hbm, name_out, desc_out,
              idx_v, rows_v, nsum_v, dsum_v, sem):
    _emb_sum_kernel(table_hbm, idx_hbm, name_out, desc_out,
                    idx_v, rows_v, nsum_v, dsum_v, sem)


def _fc_body(nsum, dsum, nlen, dlen, price, wnt, wdt, wp, bias, out):
    x1 = nsum[...] / nlen[...]
    x2 = dsum[...] / dlen[...]
    acc = jnp.dot(x1, wnt[...], preferred_element_type=jnp.float32)
    acc = acc + jnp.dot(x2, wdt[...], preferred_element_type=jnp.float32)
    out[...] = acc + price[...] * wp[...] + bias[...]


def _fc(nsum, dsum, nlen, dlen, price, wnt, wdt, wp, bias):
    grid = 8
    bb = B // grid
    return pl.pallas_call(
        _fc_body,
        grid=(grid,),
        in_specs=[
            pl.BlockSpec((bb, D), lambda i: (i, 0)),
            pl.BlockSpec((bb, D), lambda i: (i, 0)),
            pl.BlockSpec((bb, 1), lambda i: (i, 0)),
            pl.BlockSpec((bb, 1), lambda i: (i, 0)),
            pl.BlockSpec((bb, 1), lambda i: (i, 0)),
            pl.BlockSpec((D, OUT), lambda i: (0, 0)),
            pl.BlockSpec((D, OUT), lambda i: (0, 0)),
            pl.BlockSpec((1, OUT), lambda i: (0, 0)),
            pl.BlockSpec((1, OUT), lambda i: (0, 0)),
        ],
        out_specs=pl.BlockSpec((bb, OUT), lambda i: (i, 0)),
        out_shape=jax.ShapeDtypeStruct((B, OUT), jnp.float32),
    )(nsum, dsum, nlen, dlen, price, wnt, wdt, wp, bias)


def kernel(name_idxs, name_len, desc_idxs, desc_len, union_idxs, union_len,
           price, emb_table, fc_w, fc_b):
    del union_idxs, union_len
    pad = jnp.zeros((B, ROWS - L_NAME - L_DESC), jnp.int32)
    cat_idx = jnp.concatenate(
        [name_idxs.astype(jnp.int32), desc_idxs.astype(jnp.int32), pad],
        axis=1).reshape(B, 2, CHUNK)

    nsum, dsum = _emb_sums(emb_table, cat_idx)

    nlen = jnp.maximum(name_len, 1).astype(jnp.float32).reshape(B, 1)
    dlen = jnp.maximum(desc_len, 1).astype(jnp.float32).reshape(B, 1)
    wnt = fc_w[:, :D].T
    wdt = fc_w[:, D:2 * D].T
    wp = fc_w[:, 2 * D].reshape(1, OUT)
    bias = fc_b.reshape(1, OUT)
    return _fc(nsum, dsum, nlen, dlen, price.reshape(B, 1), wnt, wdt, wp, bias)
